# Initial kernel scaffold; baseline (speedup 1.0000x reference)
import jax
import jax.numpy as jnp
from jax.experimental import pallas as pl


def kernel(x, edge_index, edge_attr, batch, W_xz, b_xz, W_hz, b_hz, W_xr, b_xr, W_hr, b_hr, W_xh, b_xh, W_hh, b_hh, pool_w, lin1_W, lin1_b, lin2_W, lin2_b):
    raise NotImplementedError("write your pallas kernel here")


# speedup vs baseline: 5.2165x; 5.2165x over previous
import jax
import jax.numpy as jnp
from jax import lax
from jax.experimental import pallas as pl

_N = 10000
_DIM = 128
_K = 8000
_IDX_BITS = 14


def _body(xT_ref, WzT_ref, WhT_ref, bz_ref, bh_ref, pw_ref,
          l1WT_ref, l1b_ref, l2WT_ref, l2b_ref, out_ref):
    xT = xT_ref[...]
    az = jnp.dot(WzT_ref[...], xT, preferred_element_type=jnp.float32) + bz_ref[...]
    ah = jnp.dot(WhT_ref[...], xT, preferred_element_type=jnp.float32) + bh_ref[...]
    z = jax.nn.sigmoid(az)
    htil = jnp.tanh(ah)
    hT = jax.nn.relu((1.0 - z) * htil)

    pw = pw_ref[...]
    norm = jnp.sqrt(jnp.sum(pw * pw))
    sraw = jnp.dot(pw, hT, preferred_element_type=jnp.float32)
    sc = jnp.tanh(sraw / norm)

    bits = lax.bitcast_convert_type(sc, jnp.int32)
    key = jnp.where(bits >= 0, bits, bits ^ jnp.int32(0x7FFFFFFF))
    ub = lax.bitcast_convert_type(key ^ jnp.int32(-2147483648), jnp.uint32)

    T = jnp.uint32(0)
    for b in reversed(range(32)):
        cand = T | jnp.uint32(1 << b)
        cnt = jnp.sum((ub >= cand).astype(jnp.int32))
        T = jnp.where(cnt >= _K, cand, T)

    c_gt = jnp.sum((ub > T).astype(jnp.int32))
    m = _K - c_gt
    tie = ub == T
    idx = lax.broadcasted_iota(jnp.int32, (1, _N), 1)
    C = jnp.int32(0)
    for b in reversed(range(_IDX_BITS)):
        candc = C | jnp.int32(1 << b)
        f = jnp.sum((tie & (idx < candc)).astype(jnp.int32))
        C = jnp.where(f <= m, candc, C)

    mask = (ub > T) | (tie & (idx < C))
    xp = hT * sc
    gmax = jnp.max(jnp.where(mask, xp, -jnp.inf), axis=1, keepdims=True)
    gsum = jnp.sum(jnp.where(mask, xp, 0.0), axis=1, keepdims=True)
    gmean = gsum / jnp.float32(_K)

    g = jnp.concatenate([gmax, gmean], axis=0)
    t1 = jax.nn.relu(jnp.dot(l1WT_ref[...], g, preferred_element_type=jnp.float32)
                     + l1b_ref[...])
    out_ref[...] = (jnp.dot(l2WT_ref[...], t1, preferred_element_type=jnp.float32)
                    + l2b_ref[...])


def _run(xT, WzT, WhT, bz, bh, pw, l1WT, l1b, l2WT, l2b, *, interpret=False):
    return pl.pallas_call(
        _body,
        out_shape=jax.ShapeDtypeStruct((1, 1), jnp.float32),
        interpret=interpret,
    )(xT, WzT, WhT, bz, bh, pw, l1WT, l1b, l2WT, l2b)


def kernel(x, edge_index, edge_attr, batch, W_xz, b_xz, W_hz, b_hz,
           W_xr, b_xr, W_hr, b_hr, W_xh, b_xh, W_hh, b_hh,
           pool_w, lin1_W, lin1_b, lin2_W, lin2_b):
    xT = x.T
    bz = (b_xz + b_hz).reshape(_DIM, 1)
    bh = (b_xh + b_hh).reshape(_DIM, 1)
    return _run(xT, W_xz.T, W_xh.T, bz, bh, pool_w.reshape(1, _DIM),
                lin1_W.T, lin1_b.reshape(_DIM, 1),
                lin2_W.T, lin2_b.reshape(1, 1))


# speedup vs baseline: 6.3648x; 1.2201x over previous
import jax
import jax.numpy as jnp
from jax import lax
from jax.experimental import pallas as pl

_N = 10000
_DIM = 128
_K = 8000
_IDX_BITS = 14


_DN_T = (((1,), (1,)), ((), ()))


def _body(x_ref, WzT_ref, WhT_ref, bz_ref, bh_ref, pw_ref,
          l1WT_ref, l1b_ref, l2WT_ref, l2b_ref, out_ref):
    x = x_ref[...]
    az = lax.dot_general(WzT_ref[...], x, _DN_T,
                         preferred_element_type=jnp.float32) + bz_ref[...]
    ah = lax.dot_general(WhT_ref[...], x, _DN_T,
                         preferred_element_type=jnp.float32) + bh_ref[...]
    z = jax.nn.sigmoid(az)
    htil = jnp.tanh(ah)
    hT = jax.nn.relu((1.0 - z) * htil)

    pw = pw_ref[...]
    norm = jnp.sqrt(jnp.sum(pw * pw))
    sraw = jnp.dot(pw, hT, preferred_element_type=jnp.float32)
    sc = jnp.tanh(sraw / norm)

    bits = lax.bitcast_convert_type(sc, jnp.int32)
    key = jnp.where(bits >= 0, bits, bits ^ jnp.int32(0x7FFFFFFF))
    ub = lax.bitcast_convert_type(key ^ jnp.int32(-2147483648), jnp.uint32)

    T = jnp.uint32(0)
    for b in reversed(range(32)):
        cand = T | jnp.uint32(1 << b)
        cnt = jnp.sum((ub >= cand).astype(jnp.int32))
        T = jnp.where(cnt >= _K, cand, T)

    c_gt = jnp.sum((ub > T).astype(jnp.int32))
    m = _K - c_gt
    tie = ub == T
    idx = lax.broadcasted_iota(jnp.int32, (1, _N), 1)
    C = jnp.int32(0)
    for b in reversed(range(_IDX_BITS)):
        candc = C | jnp.int32(1 << b)
        f = jnp.sum((tie & (idx < candc)).astype(jnp.int32))
        C = jnp.where(f <= m, candc, C)

    mask = (ub > T) | (tie & (idx < C))
    scm = jnp.where(mask, sc, 0.0)
    xpm = hT * scm
    gmax = jnp.max(jnp.where(mask, xpm, -jnp.inf), axis=1, keepdims=True)
    gsum = jnp.sum(xpm, axis=1, keepdims=True)
    gmean = gsum / jnp.float32(_K)

    g = jnp.concatenate([gmax, gmean], axis=0)
    t1 = jax.nn.relu(jnp.dot(l1WT_ref[...], g, preferred_element_type=jnp.float32)
                     + l1b_ref[...])
    out_ref[...] = (jnp.dot(l2WT_ref[...], t1, preferred_element_type=jnp.float32)
                    + l2b_ref[...])


def _run(x, WzT, WhT, bz, bh, pw, l1WT, l1b, l2WT, l2b, *, interpret=False):
    return pl.pallas_call(
        _body,
        out_shape=jax.ShapeDtypeStruct((1, 1), jnp.float32),
        interpret=interpret,
    )(x, WzT, WhT, bz, bh, pw, l1WT, l1b, l2WT, l2b)


def kernel(x, edge_index, edge_attr, batch, W_xz, b_xz, W_hz, b_hz,
           W_xr, b_xr, W_hr, b_hr, W_xh, b_xh, W_hh, b_hh,
           pool_w, lin1_W, lin1_b, lin2_W, lin2_b):
    bz = (b_xz + b_hz).reshape(_DIM, 1)
    bh = (b_xh + b_hh).reshape(_DIM, 1)
    return _run(x, W_xz.T, W_xh.T, bz, bh, pool_w.reshape(1, _DIM),
                lin1_W.T, lin1_b.reshape(_DIM, 1),
                lin2_W.T, lin2_b.reshape(1, 1))


# speedup vs baseline: 6.4036x; 1.0061x over previous
import jax
import jax.numpy as jnp
from jax import lax
from jax.experimental import pallas as pl

_N = 10000
_DIM = 128
_K = 8000
_IDX_BITS = 14

_DN_T = (((1,), (1,)), ((), ()))
_DN = (((1,), (0,)), ((), ()))


def _body(x_ref, WzT_ref, WhT_ref, bxz_ref, bhz_ref, bxh_ref, bhh_ref, pw_ref,
          l1WT_ref, l1b_ref, l2WT_ref, l2b_ref, out_ref):
    x = x_ref[...]
    az = (lax.dot_general(WzT_ref[...], x, _DN_T,
                          preferred_element_type=jnp.float32)
          + bxz_ref[...]) + bhz_ref[...]
    ah = (lax.dot_general(WhT_ref[...], x, _DN_T,
                          preferred_element_type=jnp.float32)
          + bxh_ref[...]) + bhh_ref[...]
    z = jax.nn.sigmoid(az)
    htil = jnp.tanh(ah)
    hT = jax.nn.relu((1.0 - z) * htil)

    pw = pw_ref[...]
    norm = jnp.sqrt(jnp.sum(pw * pw))
    sraw = lax.dot_general(pw, hT, _DN,
                           preferred_element_type=jnp.float32)
    sc = jnp.tanh(sraw / norm)

    bits = lax.bitcast_convert_type(sc, jnp.int32)
    key = jnp.where(bits >= 0, bits, bits ^ jnp.int32(0x7FFFFFFF))
    ub = lax.bitcast_convert_type(key ^ jnp.int32(-2147483648), jnp.uint32)

    ubp = jnp.reshape(
        jnp.concatenate([ub, jnp.zeros((1, 240), jnp.uint32)], axis=1),
        (80, 128))

    T = jnp.uint32(0)
    for b in reversed(range(32)):
        cand = T | jnp.uint32(1 << b)
        cnt = jnp.sum((ubp >= cand).astype(jnp.int32))
        T = jnp.where(cnt >= _K, cand, T)

    c_gt = jnp.sum((ubp > T).astype(jnp.int32))
    m = _K - c_gt
    tiep = ubp == T
    idxp = (lax.broadcasted_iota(jnp.int32, (80, 128), 0) * 128
            + lax.broadcasted_iota(jnp.int32, (80, 128), 1))
    C = jnp.int32(0)
    for b in reversed(range(_IDX_BITS)):
        candc = C | jnp.int32(1 << b)
        f = jnp.sum((tiep & (idxp < candc)).astype(jnp.int32))
        C = jnp.where(f <= m, candc, C)

    tie = ub == T
    idx = lax.broadcasted_iota(jnp.int32, (1, _N), 1)
    mask = (ub > T) | (tie & (idx < C))
    scm = jnp.where(mask, sc, 0.0)
    xpm = hT * scm
    gmax = jnp.max(jnp.where(mask, xpm, -jnp.inf), axis=1, keepdims=True)
    gsum = jnp.sum(xpm, axis=1, keepdims=True)
    gmean = gsum / jnp.float32(_K)

    g = jnp.concatenate([gmax, gmean], axis=0)
    t1 = jax.nn.relu(lax.dot_general(l1WT_ref[...], g, _DN,
                                     preferred_element_type=jnp.float32)
                     + l1b_ref[...])
    out_ref[...] = (lax.dot_general(l2WT_ref[...], t1, _DN,
                                    preferred_element_type=jnp.float32)
                    + l2b_ref[...])


def _run(x, WzT, WhT, bxz, bhz, bxh, bhh, pw, l1WT, l1b, l2WT, l2b,
         *, interpret=False):
    return pl.pallas_call(
        _body,
        out_shape=jax.ShapeDtypeStruct((1, 1), jnp.float32),
        interpret=interpret,
    )(x, WzT, WhT, bxz, bhz, bxh, bhh, pw, l1WT, l1b, l2WT, l2b)


def kernel(x, edge_index, edge_attr, batch, W_xz, b_xz, W_hz, b_hz,
           W_xr, b_xr, W_hr, b_hr, W_xh, b_xh, W_hh, b_hh,
           pool_w, lin1_W, lin1_b, lin2_W, lin2_b):
    return _run(x, W_xz.T, W_xh.T,
                b_xz.reshape(_DIM, 1), b_hz.reshape(_DIM, 1),
                b_xh.reshape(_DIM, 1), b_hh.reshape(_DIM, 1),
                pool_w.reshape(1, _DIM),
                lin1_W.T, lin1_b.reshape(_DIM, 1),
                lin2_W.T, lin2_b.reshape(1, 1))


# speedup vs baseline: 7.2754x; 1.1362x over previous
import jax
import jax.numpy as jnp
from jax import lax
from jax.experimental import pallas as pl

_N = 10000
_DIM = 128
_K = 8000
_IDX_BITS = 14

_DN_T = (((1,), (1,)), ((), ()))
_DN = (((1,), (0,)), ((), ()))


def _body(x_ref, WzT_ref, WhT_ref, bxz_ref, bhz_ref, bxh_ref, bhh_ref, pw_ref,
          l1W_ref, l1b_ref, l2W_ref, l2b_ref, out_ref):
    x = x_ref[...]
    az = (lax.dot_general(WzT_ref[...], x, _DN_T,
                          preferred_element_type=jnp.float32)
          + bxz_ref[...]) + bhz_ref[...]
    ah = (lax.dot_general(WhT_ref[...], x, _DN_T,
                          preferred_element_type=jnp.float32)
          + bxh_ref[...]) + bhh_ref[...]
    z = jax.nn.sigmoid(az)
    htil = jnp.tanh(ah)
    hT = jax.nn.relu((1.0 - z) * htil)

    pw = pw_ref[...]
    norm = jnp.sqrt(jnp.sum(pw * pw))
    sraw = lax.dot_general(pw, hT, _DN,
                           preferred_element_type=jnp.float32)
    sc = jnp.tanh(sraw / norm)

    bits = lax.bitcast_convert_type(sc, jnp.int32)
    key = jnp.where(bits >= 0, bits, bits ^ jnp.int32(0x7FFFFFFF))
    ub = lax.bitcast_convert_type(key ^ jnp.int32(-2147483648), jnp.uint32)

    ubp = jnp.reshape(
        jnp.concatenate([ub, jnp.zeros((1, 240), jnp.uint32)], axis=1),
        (80, 128))

    T = jnp.uint32(0)
    for b in reversed(range(32)):
        cand = T | jnp.uint32(1 << b)
        cnt = jnp.sum((ubp >= cand).astype(jnp.int32))
        T = jnp.where(cnt >= _K, cand, T)

    c_gt = jnp.sum((ubp > T).astype(jnp.int32))
    m = _K - c_gt
    tiep = ubp == T
    idxp = (lax.broadcasted_iota(jnp.int32, (80, 128), 0) * 128
            + lax.broadcasted_iota(jnp.int32, (80, 128), 1))
    C = jnp.int32(0)
    for b in reversed(range(_IDX_BITS)):
        candc = C | jnp.int32(1 << b)
        f = jnp.sum((tiep & (idxp < candc)).astype(jnp.int32))
        C = jnp.where(f <= m, candc, C)

    tie = ub == T
    idx = lax.broadcasted_iota(jnp.int32, (1, _N), 1)
    mask = (ub > T) | (tie & (idx < C))
    scm = jnp.where(mask, sc, 0.0)
    xpm = hT * scm
    gmax = jnp.max(jnp.where(mask, xpm, -jnp.inf), axis=1, keepdims=True)
    gsum = jnp.sum(xpm, axis=1, keepdims=True)
    gmean = gsum / jnp.float32(_K)

    g = jnp.transpose(jnp.concatenate([gmax, gmean], axis=0),
                      (1, 0))
    t1 = jax.nn.relu(lax.dot_general(g, l1W_ref[...], _DN,
                                     preferred_element_type=jnp.float32)
                     + l1b_ref[...])
    out_ref[...] = (jnp.sum(t1 * l2W_ref[...], axis=1, keepdims=True)
                    + l2b_ref[...])


def _run(x, WzT, WhT, bxz, bhz, bxh, bhh, pw, l1W, l1b, l2W, l2b,
         *, interpret=False):
    return pl.pallas_call(
        _body,
        out_shape=jax.ShapeDtypeStruct((1, 1), jnp.float32),
        interpret=interpret,
    )(x, WzT, WhT, bxz, bhz, bxh, bhh, pw, l1W, l1b, l2W, l2b)


def kernel(x, edge_index, edge_attr, batch, W_xz, b_xz, W_hz, b_hz,
           W_xr, b_xr, W_hr, b_hr, W_xh, b_xh, W_hh, b_hh,
           pool_w, lin1_W, lin1_b, lin2_W, lin2_b):
    return _run(x, W_xz.T, W_xh.T,
                b_xz.reshape(_DIM, 1), b_hz.reshape(_DIM, 1),
                b_xh.reshape(_DIM, 1), b_hh.reshape(_DIM, 1),
                pool_w.reshape(1, _DIM),
                lin1_W, lin1_b.reshape(1, _DIM),
                lin2_W.reshape(1, _DIM), lin2_b.reshape(1, 1))


# speedup vs baseline: 8.1969x; 1.1267x over previous
import jax
import jax.numpy as jnp
from jax import lax
from jax.experimental import pallas as pl

_N = 10000
_DIM = 128
_K = 8000
_IDX_BITS = 14

_DN_T = (((1,), (1,)), ((), ()))
_DN = (((1,), (0,)), ((), ()))


def _body(x_ref, WzT_ref, WhT_ref, bxz_ref, bhz_ref, bxh_ref, bhh_ref, pw_ref,
          l1W_ref, l1b_ref, l2W_ref, l2b_ref, out_ref):
    x = x_ref[...]
    az = (lax.dot_general(WzT_ref[...], x, _DN_T,
                          preferred_element_type=jnp.float32)
          + bxz_ref[...]) + bhz_ref[...]
    ah = (lax.dot_general(WhT_ref[...], x, _DN_T,
                          preferred_element_type=jnp.float32)
          + bxh_ref[...]) + bhh_ref[...]
    z = jax.nn.sigmoid(az)
    htil = jnp.tanh(ah)
    hT = jax.nn.relu((1.0 - z) * htil)

    pw = pw_ref[...]
    norm = jnp.sqrt(jnp.sum(pw * pw))
    sraw = lax.dot_general(pw, hT, _DN,
                           preferred_element_type=jnp.float32)
    sc = jnp.tanh(sraw / norm)

    bits = lax.bitcast_convert_type(sc, jnp.int32)
    key = jnp.where(bits >= 0, bits, bits ^ jnp.int32(0x7FFFFFFF))
    ub = lax.bitcast_convert_type(key ^ jnp.int32(-2147483648), jnp.uint32)

    ubp = jnp.reshape(
        jnp.concatenate([ub, jnp.zeros((1, 240), jnp.uint32)], axis=1),
        (80, 128))

    T = jnp.uint32(0)
    for hi in range(31, -1, -2):
        b1 = jnp.uint32(1 << hi)
        b2 = jnp.uint32(1 << (hi - 1))
        c_hi = jnp.sum((ubp >= (T | b1)).astype(jnp.int32))
        c_hi2 = jnp.sum((ubp >= (T | b1 | b2)).astype(jnp.int32))
        c_lo = jnp.sum((ubp >= (T | b2)).astype(jnp.int32))
        T = jnp.where(c_hi >= _K,
                      jnp.where(c_hi2 >= _K, T | b1 | b2, T | b1),
                      jnp.where(c_lo >= _K, T | b2, T))

    c_gt = jnp.sum((ubp > T).astype(jnp.int32))
    m = _K - c_gt
    tiep = ubp == T
    idxp = (lax.broadcasted_iota(jnp.int32, (80, 128), 0) * 128
            + lax.broadcasted_iota(jnp.int32, (80, 128), 1))
    C = jnp.int32(0)
    for hi in range(_IDX_BITS - 1, -1, -2):
        b1 = jnp.int32(1 << hi)
        b2 = jnp.int32(1 << (hi - 1))
        f_hi = jnp.sum((tiep & (idxp < (C | b1))).astype(jnp.int32))
        f_hi2 = jnp.sum((tiep & (idxp < (C | b1 | b2))).astype(jnp.int32))
        f_lo = jnp.sum((tiep & (idxp < (C | b2))).astype(jnp.int32))
        C = jnp.where(f_hi <= m,
                      jnp.where(f_hi2 <= m, C | b1 | b2, C | b1),
                      jnp.where(f_lo <= m, C | b2, C))

    tie = ub == T
    idx = lax.broadcasted_iota(jnp.int32, (1, _N), 1)
    mask = (ub > T) | (tie & (idx < C))
    scm = jnp.where(mask, sc, 0.0)
    xpm = hT * scm
    gmax = jnp.max(jnp.where(mask, xpm, -jnp.inf), axis=1, keepdims=True)
    gsum = jnp.sum(xpm, axis=1, keepdims=True)
    gmean = gsum / jnp.float32(_K)

    g = jnp.transpose(jnp.concatenate([gmax, gmean], axis=0),
                      (1, 0))
    t1 = jax.nn.relu(lax.dot_general(g, l1W_ref[...], _DN,
                                     preferred_element_type=jnp.float32)
                     + l1b_ref[...])
    out_ref[...] = (jnp.sum(t1 * l2W_ref[...], axis=1, keepdims=True)
                    + l2b_ref[...])


def _run(x, WzT, WhT, bxz, bhz, bxh, bhh, pw, l1W, l1b, l2W, l2b,
         *, interpret=False):
    return pl.pallas_call(
        _body,
        out_shape=jax.ShapeDtypeStruct((1, 1), jnp.float32),
        interpret=interpret,
    )(x, WzT, WhT, bxz, bhz, bxh, bhh, pw, l1W, l1b, l2W, l2b)


def kernel(x, edge_index, edge_attr, batch, W_xz, b_xz, W_hz, b_hz,
           W_xr, b_xr, W_hr, b_hr, W_xh, b_xh, W_hh, b_hh,
           pool_w, lin1_W, lin1_b, lin2_W, lin2_b):
    return _run(x, W_xz.T, W_xh.T,
                b_xz.reshape(_DIM, 1), b_hz.reshape(_DIM, 1),
                b_xh.reshape(_DIM, 1), b_hh.reshape(_DIM, 1),
                pool_w.reshape(1, _DIM),
                lin1_W, lin1_b.reshape(1, _DIM),
                lin2_W.reshape(1, _DIM), lin2_b.reshape(1, 1))
